# baseline (speedup 1.0000x reference)
import functools

import jax
import jax.numpy as jnp
from jax import lax
from jax.experimental import pallas as pl
from jax.experimental.pallas import tpu as pltpu
from jax.experimental.pallas import tpu_sc as plsc

N = 10000
E = 640000
B = 64
NA = 10240
NW = 32
C = 128
K0 = 158
K1 = 158
KM = max(K0, K1)
EP = 16 * C * (K0 + K1)
ROWS_PER_TILE = NA // 16
R = 1024
GRID = NA // R

@functools.lru_cache(maxsize=None)
def _mesh():
    return plsc.VectorSubcoreMesh(core_axis_name="c", subcore_axis_name="s")


_SC_PARAMS = pltpu.CompilerParams(use_tc_tiling_on_sc=False)


def _zero_rows(zbuf, di):
    @pl.loop(0, C)
    def _(r):
        for j in range(di // 16):
            zbuf.at[pl.ds(r, 1), pl.ds(j * 16, 16)][...] = jnp.zeros(
                (1, 16), jnp.float32)


@functools.lru_cache(maxsize=None)
def _make_agg(di):
    @functools.partial(
        pl.kernel,
        out_type=jax.ShapeDtypeStruct((2, NA, di), jnp.float32),
        mesh=_mesh(),
        compiler_params=_SC_PARAMS,
        scratch_types=(
            [pltpu.VMEM((KM, C), jnp.int32),
             pltpu.VMEM((KM, C), jnp.int32)]
            + [pltpu.VMEM((C, di), jnp.float32),
               pltpu.VMEM((C, di), jnp.float32),
               pltpu.VMEM_SHARED((NA, di), jnp.float32)]
        ),
    )
    def agg(table_hbm, src_hbm, dst_hbm, out_hbm, sidx, didx, rows, zbuf,
            acc):
        c = lax.axis_index("c")
        s = lax.axis_index("s")
        w = c * 16 + s
        _zero_rows(zbuf, di)
        for j in range(ROWS_PER_TILE // 128):
            pltpu.sync_copy(zbuf,
                            acc.at[pl.ds(s * ROWS_PER_TILE + j * 128, 128)])
        pltpu.sync_copy(src_hbm.at[w], sidx)
        pltpu.sync_copy(dst_hbm.at[w], didx)
        plsc.subcore_barrier()

        def run(kk):
            @pl.loop(0, kk)
            def _(i):
                pltpu.sync_copy(table_hbm.at[sidx.at[i]], rows)
                pltpu.sync_copy(rows, acc.at[didx.at[i]], add=True)

        if K0 == K1:
            run(K0)
        else:
            @pl.when(c == 0)
            def _():
                run(K0)

            @pl.when(c == 1)
            def _():
                run(K1)

        plsc.subcore_barrier()
        sl = pl.ds(s * ROWS_PER_TILE, ROWS_PER_TILE)
        pltpu.sync_copy(acc.at[sl], out_hbm.at[c, sl])

    return agg


@functools.lru_cache(maxsize=None)
def _make_deg():
    di = 16

    @functools.partial(
        pl.kernel,
        out_type=jax.ShapeDtypeStruct((2, NA, di), jnp.float32),
        mesh=_mesh(),
        compiler_params=_SC_PARAMS,
        scratch_types=(
            [pltpu.VMEM((KM, C), jnp.int32),
             pltpu.VMEM((C, di), jnp.float32),
             pltpu.VMEM((C, di), jnp.float32),
             pltpu.VMEM_SHARED((NA, di), jnp.float32)]
        ),
    )
    def deg(ones_hbm, dst_hbm, out_hbm, didx, ones_v, zbuf, acc):
        c = lax.axis_index("c")
        s = lax.axis_index("s")
        w = c * 16 + s
        _zero_rows(zbuf, di)
        for j in range(ROWS_PER_TILE // 128):
            pltpu.sync_copy(zbuf,
                            acc.at[pl.ds(s * ROWS_PER_TILE + j * 128, 128)])
        pltpu.sync_copy(ones_hbm, ones_v)
        pltpu.sync_copy(dst_hbm.at[w], didx)
        plsc.subcore_barrier()

        def run(kk):
            @pl.loop(0, kk)
            def _(i):
                pltpu.sync_copy(ones_v, acc.at[didx.at[i]], add=True)

        if K0 == K1:
            run(K0)
        else:
            @pl.when(c == 0)
            def _():
                run(K0)

            @pl.when(c == 1)
            def _():
                run(K1)

        plsc.subcore_barrier()
        sl = pl.ds(s * ROWS_PER_TILE, ROWS_PER_TILE)
        pltpu.sync_copy(acc.at[sl], out_hbm.at[c, sl])

    return deg


def _row_spec(width):
    return pl.BlockSpec((R, width), lambda i: (i, 0))


def _part_spec(width):
    return pl.BlockSpec((2, R, width), lambda i: (0, i, 0))


def _full_spec(a, b):
    return pl.BlockSpec((a, b), lambda i: (0, 0))


def _prep_body(degp, x, w1, t1, dinv):
    d = degp[0, :, 0:1] + degp[1, :, 0:1] + 1.0
    di = lax.rsqrt(d)
    h = jnp.dot(x[...], w1[...], preferred_element_type=jnp.float32)
    t1[...] = di * h
    dinv[...] = di


def _tc_prep(degp, x, w1):
    return pl.pallas_call(
        _prep_body,
        grid=(GRID,),
        in_specs=[_part_spec(16), _row_spec(2), _full_spec(2, 16)],
        out_specs=[_row_spec(16), _row_spec(1)],
        out_shape=[jax.ShapeDtypeStruct((NA, 16), jnp.float32),
                   jax.ShapeDtypeStruct((NA, 1), jnp.float32)],
    )(degp, x, w1)


def _mid1_body(p, t, dinv, b, t2):
    g = dinv[...] * (p[0] + p[1] + t[...])
    t2[...] = dinv[...] * jnp.maximum(g + b[...], 0.0)


def _tc_mid1(p, t, dinv, b):
    return pl.pallas_call(
        _mid1_body,
        grid=(GRID,),
        in_specs=[_part_spec(16), _row_spec(16), _row_spec(1),
                  _full_spec(1, 16)],
        out_specs=_row_spec(16),
        out_shape=jax.ShapeDtypeStruct((NA, 16), jnp.float32),
    )(p, t, dinv, b)


def _mid_body(p, t, dinv, w, b, tn):
    g = dinv[...] * (p[0] + p[1] + t[...])
    h = jnp.dot(g, w[...], preferred_element_type=jnp.float32)
    tn[...] = dinv[...] * jnp.maximum(h + b[...], 0.0)


def _tc_mid(p, t, dinv, w, b):
    din, dout = w.shape
    return pl.pallas_call(
        _mid_body,
        grid=(GRID,),
        in_specs=[_part_spec(din), _row_spec(din), _row_spec(1),
                  _full_spec(din, dout), _full_spec(1, dout)],
        out_specs=_row_spec(dout),
        out_shape=jax.ShapeDtypeStruct((NA, dout), jnp.float32),
    )(p, t, dinv, w, b)


def _final_body(p, t, dinv, bat, w6, b6, wl1, bl1, wl2, bl2, out, pool):
    i = pl.program_id(0)

    @pl.when(i == 0)
    def _():
        pool[...] = jnp.full((B, 128), -jnp.inf, jnp.float32)

    g = dinv[...] * (p[0] + p[1] + t[...])
    h = jnp.maximum(
        jnp.dot(g, w6[...], preferred_element_type=jnp.float32) + b6[...], 0.0)
    bb = bat[...]
    for seg in range(B):
        col = jnp.max(jnp.where(bb == seg, h, -jnp.inf), axis=0,
                      keepdims=True)
        pool.at[pl.ds(seg, 1), :][...] = jnp.maximum(
            pool.at[pl.ds(seg, 1), :][...], col)

    @pl.when(i == GRID - 1)
    def _():
        pd = pool[...]
        pd = jnp.where(jnp.isfinite(pd), pd, 0.0)
        hm = jnp.maximum(
            jnp.dot(pd, wl1[...], preferred_element_type=jnp.float32)
            + bl1[...], 0.0)
        out[...] = (jnp.dot(hm, wl2[...], preferred_element_type=jnp.float32)
                    + bl2[...])


def _tc_final(p, t, dinv, bat, w6, b6, wl1, bl1, wl2, bl2):
    return pl.pallas_call(
        _final_body,
        grid=(GRID,),
        in_specs=[_part_spec(96), _row_spec(96), _row_spec(1),
                  pl.BlockSpec((R, 1), lambda i: (i, 0)),
                  _full_spec(96, 128), _full_spec(1, 128),
                  _full_spec(128, 64), _full_spec(1, 64),
                  _full_spec(64, 10), _full_spec(1, 10)],
        out_specs=_full_spec(B, 10),
        out_shape=jax.ShapeDtypeStruct((B, 10), jnp.float32),
        scratch_shapes=[pltpu.VMEM((B, 128), jnp.float32)],
    )(p, t, dinv, bat, w6, b6, wl1, bl1, wl2, bl2)


def kernel(x, edge_index, batch, W1, b1, W2, b2, W3, b3, W4, b4, W5, b5,
           W6, b6, Wl1, bl1, Wl2, bl2):
    pad_e = EP - E
    src = jnp.concatenate(
        [edge_index[0], jnp.zeros((pad_e,), jnp.int32)]).reshape(NW, KM, C)
    dst = jnp.concatenate(
        [edge_index[1], jnp.full((pad_e,), N, jnp.int32)]).reshape(NW, KM, C)
    xp = jnp.concatenate([x, jnp.zeros((NA - N, 2), jnp.float32)])
    batp = jnp.concatenate(
        [batch, jnp.full((NA - N,), B, jnp.int32)]).reshape(NA, 1)
    ones = jnp.ones((C, 16), jnp.float32)

    degp = _make_deg()(ones, dst)
    t1, dinv = _tc_prep(degp, xp, W1)

    p1 = _make_agg(16)(t1, src, dst)
    t2 = _tc_mid1(p1, t1, dinv, b1.reshape(1, 16))
    p2 = _make_agg(16)(t2, src, dst)
    t3 = _tc_mid(p2, t2, dinv, W2, b2.reshape(1, -1))
    p3 = _make_agg(32)(t3, src, dst)
    t4 = _tc_mid(p3, t3, dinv, W3, b3.reshape(1, -1))
    p4 = _make_agg(48)(t4, src, dst)
    t5 = _tc_mid(p4, t4, dinv, W4, b4.reshape(1, -1))
    p5 = _make_agg(64)(t5, src, dst)
    t6 = _tc_mid(p5, t5, dinv, W5, b5.reshape(1, -1))
    p6 = _make_agg(96)(t6, src, dst)

    return _tc_final(p6, t6, dinv, batp, W6, b6.reshape(1, 128),
                     Wl1, bl1.reshape(1, 64), Wl2, bl2.reshape(1, 10))


# speedup vs baseline: 1.2417x; 1.2417x over previous
import functools

import jax
import jax.numpy as jnp
from jax import lax
from jax.experimental import pallas as pl
from jax.experimental.pallas import tpu as pltpu
from jax.experimental.pallas import tpu_sc as plsc

N = 10000
E = 640000
B = 64
NA = 10240
NW = 32
C = 128
K0 = 157
K1 = 157
KM = max(K0, K1)
EP = 16 * C * (K0 + K1)
ROWS_PER_TILE = NA // 16
R = 1024
GRID = NA // R

@functools.lru_cache(maxsize=None)
def _mesh():
    return plsc.VectorSubcoreMesh(core_axis_name="c", subcore_axis_name="s")


_SC_PARAMS = pltpu.CompilerParams(use_tc_tiling_on_sc=False)


def _zero_rows(zbuf, di):
    @pl.loop(0, C)
    def _(r):
        for j in range(di // 16):
            zbuf.at[pl.ds(r, 1), pl.ds(j * 16, 16)][...] = jnp.zeros(
                (1, 16), jnp.float32)


@functools.lru_cache(maxsize=None)
def _make_agg(di):
    @functools.partial(
        pl.kernel,
        out_type=jax.ShapeDtypeStruct((2, NA, di), jnp.float32),
        mesh=_mesh(),
        compiler_params=_SC_PARAMS,
        scratch_types=(
            [pltpu.VMEM((KM, C), jnp.int32),
             pltpu.VMEM((KM, C), jnp.int32)]
            + [pltpu.VMEM((C, di), jnp.float32),
               pltpu.VMEM((C, di), jnp.float32),
               pltpu.VMEM_SHARED((NA, di), jnp.float32)]
        ),
    )
    def agg(table_hbm, src_hbm, dst_hbm, out_hbm, sidx, didx, rows, zbuf,
            acc):
        c = lax.axis_index("c")
        s = lax.axis_index("s")
        w = c * 16 + s
        _zero_rows(zbuf, di)
        for j in range(ROWS_PER_TILE // 128):
            pltpu.sync_copy(zbuf,
                            acc.at[pl.ds(s * ROWS_PER_TILE + j * 128, 128)])
        pltpu.sync_copy(src_hbm.at[w], sidx)
        pltpu.sync_copy(dst_hbm.at[w], didx)
        plsc.subcore_barrier()

        def run(kk):
            @pl.loop(0, kk)
            def _(i):
                pltpu.sync_copy(table_hbm.at[sidx.at[i]], rows)
                pltpu.sync_copy(rows, acc.at[didx.at[i]], add=True)

        if K0 == K1:
            run(K0)
        else:
            @pl.when(c == 0)
            def _():
                run(K0)

            @pl.when(c == 1)
            def _():
                run(K1)

        plsc.subcore_barrier()
        sl = pl.ds(s * ROWS_PER_TILE, ROWS_PER_TILE)
        pltpu.sync_copy(acc.at[sl], out_hbm.at[c, sl])

    return agg


@functools.lru_cache(maxsize=None)
def _make_deg():
    di = 16

    @functools.partial(
        pl.kernel,
        out_type=jax.ShapeDtypeStruct((2, NA, di), jnp.float32),
        mesh=_mesh(),
        compiler_params=_SC_PARAMS,
        scratch_types=(
            [pltpu.VMEM((KM, C), jnp.int32),
             pltpu.VMEM((C, di), jnp.float32),
             pltpu.VMEM((C, di), jnp.float32),
             pltpu.VMEM_SHARED((NA, di), jnp.float32)]
        ),
    )
    def deg(ones_hbm, dst_hbm, out_hbm, didx, ones_v, zbuf, acc):
        c = lax.axis_index("c")
        s = lax.axis_index("s")
        w = c * 16 + s
        _zero_rows(zbuf, di)
        for j in range(ROWS_PER_TILE // 128):
            pltpu.sync_copy(zbuf,
                            acc.at[pl.ds(s * ROWS_PER_TILE + j * 128, 128)])
        pltpu.sync_copy(ones_hbm, ones_v)
        pltpu.sync_copy(dst_hbm.at[w], didx)
        plsc.subcore_barrier()

        def run(kk):
            @pl.loop(0, kk)
            def _(i):
                pltpu.sync_copy(ones_v, acc.at[didx.at[i]], add=True)

        if K0 == K1:
            run(K0)
        else:
            @pl.when(c == 0)
            def _():
                run(K0)

            @pl.when(c == 1)
            def _():
                run(K1)

        plsc.subcore_barrier()
        sl = pl.ds(s * ROWS_PER_TILE, ROWS_PER_TILE)
        pltpu.sync_copy(acc.at[sl], out_hbm.at[c, sl])

    return deg


def _row_spec(width):
    return pl.BlockSpec((R, width), lambda i: (i, 0))


def _part_spec(width):
    return pl.BlockSpec((2, R, width), lambda i: (0, i, 0))


def _full_spec(a, b):
    return pl.BlockSpec((a, b), lambda i: (0, 0))


def _prep_body(degp, x, w1, t1, dinv):
    d = degp[0, :, 0:1] + degp[1, :, 0:1] + 1.0
    di = lax.rsqrt(d)
    h = jnp.dot(x[...], w1[...], preferred_element_type=jnp.float32)
    t1[...] = di * h
    dinv[...] = di


def _tc_prep(degp, x, w1):
    return pl.pallas_call(
        _prep_body,
        grid=(GRID,),
        in_specs=[_part_spec(16), _row_spec(2), _full_spec(2, 16)],
        out_specs=[_row_spec(16), _row_spec(1)],
        out_shape=[jax.ShapeDtypeStruct((NA, 16), jnp.float32),
                   jax.ShapeDtypeStruct((NA, 1), jnp.float32)],
    )(degp, x, w1)


def _mid1_body(p, t, dinv, b, t2):
    g = dinv[...] * (p[0] + p[1] + t[...])
    t2[...] = dinv[...] * jnp.maximum(g + b[...], 0.0)


def _tc_mid1(p, t, dinv, b):
    return pl.pallas_call(
        _mid1_body,
        grid=(GRID,),
        in_specs=[_part_spec(16), _row_spec(16), _row_spec(1),
                  _full_spec(1, 16)],
        out_specs=_row_spec(16),
        out_shape=jax.ShapeDtypeStruct((NA, 16), jnp.float32),
    )(p, t, dinv, b)


def _mid_body(p, t, dinv, w, b, tn):
    g = dinv[...] * (p[0] + p[1] + t[...])
    h = jnp.dot(g, w[...], preferred_element_type=jnp.float32)
    tn[...] = dinv[...] * jnp.maximum(h + b[...], 0.0)


def _tc_mid(p, t, dinv, w, b):
    din, dout = w.shape
    return pl.pallas_call(
        _mid_body,
        grid=(GRID,),
        in_specs=[_part_spec(din), _row_spec(din), _row_spec(1),
                  _full_spec(din, dout), _full_spec(1, dout)],
        out_specs=_row_spec(dout),
        out_shape=jax.ShapeDtypeStruct((NA, dout), jnp.float32),
    )(p, t, dinv, w, b)


def _final_body(p, t, dinv, bat, w6, b6, wl1, bl1, wl2, bl2, out, pool):
    i = pl.program_id(0)

    @pl.when(i == 0)
    def _():
        pool[...] = jnp.full((B, 128), -jnp.inf, jnp.float32)

    g = dinv[...] * (p[0] + p[1] + t[...])
    h = jnp.maximum(
        jnp.dot(g, w6[...], preferred_element_type=jnp.float32) + b6[...], 0.0)
    bb = bat[...]
    for seg in range(B):
        col = jnp.max(jnp.where(bb == seg, h, -jnp.inf), axis=0,
                      keepdims=True)
        pool.at[pl.ds(seg, 1), :][...] = jnp.maximum(
            pool.at[pl.ds(seg, 1), :][...], col)

    @pl.when(i == GRID - 1)
    def _():
        pd = pool[...]
        pd = jnp.where(jnp.isfinite(pd), pd, 0.0)
        hm = jnp.maximum(
            jnp.dot(pd, wl1[...], preferred_element_type=jnp.float32)
            + bl1[...], 0.0)
        out[...] = (jnp.dot(hm, wl2[...], preferred_element_type=jnp.float32)
                    + bl2[...])


def _tc_final(p, t, dinv, bat, w6, b6, wl1, bl1, wl2, bl2):
    return pl.pallas_call(
        _final_body,
        grid=(GRID,),
        in_specs=[_part_spec(96), _row_spec(96), _row_spec(1),
                  pl.BlockSpec((R, 1), lambda i: (i, 0)),
                  _full_spec(96, 128), _full_spec(1, 128),
                  _full_spec(128, 64), _full_spec(1, 64),
                  _full_spec(64, 10), _full_spec(1, 10)],
        out_specs=_full_spec(B, 10),
        out_shape=jax.ShapeDtypeStruct((B, 10), jnp.float32),
        scratch_shapes=[pltpu.VMEM((B, 128), jnp.float32)],
    )(p, t, dinv, bat, w6, b6, wl1, bl1, wl2, bl2)


def kernel(x, edge_index, batch, W1, b1, W2, b2, W3, b3, W4, b4, W5, b5,
           W6, b6, Wl1, bl1, Wl2, bl2):
    pad_e = EP - E
    src = jnp.concatenate(
        [edge_index[0], jnp.zeros((pad_e,), jnp.int32)]).reshape(NW, KM, C)
    dst = jnp.concatenate(
        [edge_index[1], jnp.full((pad_e,), N, jnp.int32)]).reshape(NW, KM, C)
    xp = jnp.concatenate([x, jnp.zeros((NA - N, 2), jnp.float32)])
    batp = jnp.concatenate(
        [batch, jnp.full((NA - N,), B, jnp.int32)]).reshape(NA, 1)
    ones = jnp.ones((C, 16), jnp.float32)

    degp = _make_deg()(ones, dst)
    t1, dinv = _tc_prep(degp, xp, W1)

    p1 = _make_agg(16)(t1, src, dst)
    t2 = _tc_mid1(p1, t1, dinv, b1.reshape(1, 16))
    p2 = _make_agg(16)(t2, src, dst)
    t3 = _tc_mid(p2, t2, dinv, W2, b2.reshape(1, -1))
    p3 = _make_agg(32)(t3, src, dst)
    t4 = _tc_mid(p3, t3, dinv, W3, b3.reshape(1, -1))
    p4 = _make_agg(48)(t4, src, dst)
    t5 = _tc_mid(p4, t4, dinv, W4, b4.reshape(1, -1))
    p5 = _make_agg(64)(t5, src, dst)
    t6 = _tc_mid(p5, t5, dinv, W5, b5.reshape(1, -1))
    p6 = _make_agg(96)(t6, src, dst)

    return _tc_final(p6, t6, dinv, batp, W6, b6.reshape(1, 128),
                     Wl1, bl1.reshape(1, 64), Wl2, bl2.reshape(1, 10))


# speedup vs baseline: 1.4398x; 1.1595x over previous
import functools

import jax
import jax.numpy as jnp
from jax import lax
from jax.experimental import pallas as pl
from jax.experimental.pallas import tpu as pltpu
from jax.experimental.pallas import tpu_sc as plsc

N = 10000
E = 640000
B = 64
NA = 10240
NW = 32
C = 128
K0 = 157
K1 = 157
KM = max(K0, K1)
EP = 16 * C * (K0 + K1)
ROWS_PER_TILE = NA // 16
R = 1024
GRID = NA // R

@functools.lru_cache(maxsize=None)
def _mesh():
    return plsc.VectorSubcoreMesh(core_axis_name="c", subcore_axis_name="s")


_SC_PARAMS = pltpu.CompilerParams(use_tc_tiling_on_sc=False)


def _zero_rows(zbuf, di):
    @pl.loop(0, C)
    def _(r):
        for j in range(di // 16):
            zbuf.at[pl.ds(r, 1), pl.ds(j * 16, 16)][...] = jnp.zeros(
                (1, 16), jnp.float32)


@functools.lru_cache(maxsize=None)
def _make_agg(di):
    @functools.partial(
        pl.kernel,
        out_type=jax.ShapeDtypeStruct((2, NA, di), jnp.float32),
        mesh=_mesh(),
        compiler_params=_SC_PARAMS,
        scratch_types=(
            [pltpu.VMEM((KM, C), jnp.int32),
             pltpu.VMEM((KM, C), jnp.int32)]
            + [pltpu.VMEM((C, di), jnp.float32),
               pltpu.VMEM((C, di), jnp.float32),
               pltpu.VMEM_SHARED((NA, di), jnp.float32)]
            + [pltpu.SemaphoreType.DMA] * 2
        ),
    )
    def agg(table_hbm, src_hbm, dst_hbm, out_hbm, sidx, didx, r0, r1,
            acc, s0, s1):
        rows = (r0, r1)
        ss = (s0, s1)
        c = lax.axis_index("c")
        s = lax.axis_index("s")
        w = c * 16 + s
        _zero_rows(r0, di)
        for j in range(ROWS_PER_TILE // 128):
            pltpu.sync_copy(r0,
                            acc.at[pl.ds(s * ROWS_PER_TILE + j * 128, 128)])
        pltpu.sync_copy(src_hbm.at[w], sidx)
        pltpu.sync_copy(dst_hbm.at[w], didx)
        plsc.subcore_barrier()
        pltpu.sync_copy(table_hbm.at[sidx.at[0]], rows[0])
        pltpu.sync_copy(rows[0], acc.at[didx.at[0]], add=True)
        for b in range(2):
            pltpu.sync_copy(table_hbm.at[sidx.at[1 + b]], rows[b])
            pltpu.async_copy(rows[b], acc.at[didx.at[1 + b]], ss[b],
                             add=True)

        @pl.loop(1, (K0 - 1) // 2)
        def _(j):
            for b in range(2):
                i = 2 * j + 1 + b
                pltpu.make_async_copy(
                    rows[b], acc.at[didx.at[i]], ss[b]).wait()
                pltpu.sync_copy(table_hbm.at[sidx.at[i]], rows[b])
                pltpu.async_copy(rows[b], acc.at[didx.at[i]], ss[b],
                                 add=True)

        for b in range(2):
            pltpu.make_async_copy(
                rows[b], acc.at[didx.at[K0 - 2 + b]], ss[b]).wait()
        plsc.subcore_barrier()
        sl = pl.ds(s * ROWS_PER_TILE, ROWS_PER_TILE)
        pltpu.sync_copy(acc.at[sl], out_hbm.at[c, sl])

    return agg


@functools.lru_cache(maxsize=None)
def _make_deg():
    di = 16

    @functools.partial(
        pl.kernel,
        out_type=jax.ShapeDtypeStruct((2, NA, di), jnp.float32),
        mesh=_mesh(),
        compiler_params=_SC_PARAMS,
        scratch_types=(
            [pltpu.VMEM((KM, C), jnp.int32),
             pltpu.VMEM((C, di), jnp.float32),
             pltpu.VMEM((C, di), jnp.float32),
             pltpu.VMEM_SHARED((NA, di), jnp.float32)]
        ),
    )
    def deg(ones_hbm, dst_hbm, out_hbm, didx, ones_v, zbuf, acc):
        c = lax.axis_index("c")
        s = lax.axis_index("s")
        w = c * 16 + s
        _zero_rows(zbuf, di)
        for j in range(ROWS_PER_TILE // 128):
            pltpu.sync_copy(zbuf,
                            acc.at[pl.ds(s * ROWS_PER_TILE + j * 128, 128)])
        pltpu.sync_copy(ones_hbm, ones_v)
        pltpu.sync_copy(dst_hbm.at[w], didx)
        plsc.subcore_barrier()

        def run(kk):
            @pl.loop(0, kk)
            def _(i):
                pltpu.sync_copy(ones_v, acc.at[didx.at[i]], add=True)

        if K0 == K1:
            run(K0)
        else:
            @pl.when(c == 0)
            def _():
                run(K0)

            @pl.when(c == 1)
            def _():
                run(K1)

        plsc.subcore_barrier()
        sl = pl.ds(s * ROWS_PER_TILE, ROWS_PER_TILE)
        pltpu.sync_copy(acc.at[sl], out_hbm.at[c, sl])

    return deg


def _row_spec(width):
    return pl.BlockSpec((R, width), lambda i: (i, 0))


def _part_spec(width):
    return pl.BlockSpec((2, R, width), lambda i: (0, i, 0))


def _full_spec(a, b):
    return pl.BlockSpec((a, b), lambda i: (0, 0))


def _prep_body(degp, x, w1, t1, dinv):
    d = degp[0, :, 0:1] + degp[1, :, 0:1] + 1.0
    di = lax.rsqrt(d)
    h = jnp.dot(x[...], w1[...], preferred_element_type=jnp.float32)
    t1[...] = di * h
    dinv[...] = di


def _tc_prep(degp, x, w1):
    return pl.pallas_call(
        _prep_body,
        grid=(GRID,),
        in_specs=[_part_spec(16), _row_spec(2), _full_spec(2, 16)],
        out_specs=[_row_spec(16), _row_spec(1)],
        out_shape=[jax.ShapeDtypeStruct((NA, 16), jnp.float32),
                   jax.ShapeDtypeStruct((NA, 1), jnp.float32)],
    )(degp, x, w1)


def _mid1_body(p, t, dinv, b, t2):
    g = dinv[...] * (p[0] + p[1] + t[...])
    t2[...] = dinv[...] * jnp.maximum(g + b[...], 0.0)


def _tc_mid1(p, t, dinv, b):
    return pl.pallas_call(
        _mid1_body,
        grid=(GRID,),
        in_specs=[_part_spec(16), _row_spec(16), _row_spec(1),
                  _full_spec(1, 16)],
        out_specs=_row_spec(16),
        out_shape=jax.ShapeDtypeStruct((NA, 16), jnp.float32),
    )(p, t, dinv, b)


def _mid_body(p, t, dinv, w, b, tn):
    g = dinv[...] * (p[0] + p[1] + t[...])
    h = jnp.dot(g, w[...], preferred_element_type=jnp.float32)
    tn[...] = dinv[...] * jnp.maximum(h + b[...], 0.0)


def _tc_mid(p, t, dinv, w, b):
    din, dout = w.shape
    return pl.pallas_call(
        _mid_body,
        grid=(GRID,),
        in_specs=[_part_spec(din), _row_spec(din), _row_spec(1),
                  _full_spec(din, dout), _full_spec(1, dout)],
        out_specs=_row_spec(dout),
        out_shape=jax.ShapeDtypeStruct((NA, dout), jnp.float32),
    )(p, t, dinv, w, b)


def _final_body(p, t, dinv, bat, w6, b6, wl1, bl1, wl2, bl2, out, pool):
    i = pl.program_id(0)

    @pl.when(i == 0)
    def _():
        pool[...] = jnp.full((B, 128), -jnp.inf, jnp.float32)

    g = dinv[...] * (p[0] + p[1] + t[...])
    h = jnp.maximum(
        jnp.dot(g, w6[...], preferred_element_type=jnp.float32) + b6[...], 0.0)
    bb = bat[...]
    for seg in range(B):
        col = jnp.max(jnp.where(bb == seg, h, -jnp.inf), axis=0,
                      keepdims=True)
        pool.at[pl.ds(seg, 1), :][...] = jnp.maximum(
            pool.at[pl.ds(seg, 1), :][...], col)

    @pl.when(i == GRID - 1)
    def _():
        pd = pool[...]
        pd = jnp.where(jnp.isfinite(pd), pd, 0.0)
        hm = jnp.maximum(
            jnp.dot(pd, wl1[...], preferred_element_type=jnp.float32)
            + bl1[...], 0.0)
        out[...] = (jnp.dot(hm, wl2[...], preferred_element_type=jnp.float32)
                    + bl2[...])


def _tc_final(p, t, dinv, bat, w6, b6, wl1, bl1, wl2, bl2):
    return pl.pallas_call(
        _final_body,
        grid=(GRID,),
        in_specs=[_part_spec(96), _row_spec(96), _row_spec(1),
                  pl.BlockSpec((R, 1), lambda i: (i, 0)),
                  _full_spec(96, 128), _full_spec(1, 128),
                  _full_spec(128, 64), _full_spec(1, 64),
                  _full_spec(64, 10), _full_spec(1, 10)],
        out_specs=_full_spec(B, 10),
        out_shape=jax.ShapeDtypeStruct((B, 10), jnp.float32),
        scratch_shapes=[pltpu.VMEM((B, 128), jnp.float32)],
    )(p, t, dinv, bat, w6, b6, wl1, bl1, wl2, bl2)


def kernel(x, edge_index, batch, W1, b1, W2, b2, W3, b3, W4, b4, W5, b5,
           W6, b6, Wl1, bl1, Wl2, bl2):
    pad_e = EP - E
    src = jnp.concatenate(
        [edge_index[0], jnp.zeros((pad_e,), jnp.int32)]).reshape(NW, KM, C)
    dst = jnp.concatenate(
        [edge_index[1], jnp.full((pad_e,), N, jnp.int32)]).reshape(NW, KM, C)
    xp = jnp.concatenate([x, jnp.zeros((NA - N, 2), jnp.float32)])
    batp = jnp.concatenate(
        [batch, jnp.full((NA - N,), B, jnp.int32)]).reshape(NA, 1)
    ones = jnp.ones((C, 16), jnp.float32)

    degp = _make_deg()(ones, dst)
    t1, dinv = _tc_prep(degp, xp, W1)

    p1 = _make_agg(16)(t1, src, dst)
    t2 = _tc_mid1(p1, t1, dinv, b1.reshape(1, 16))
    p2 = _make_agg(16)(t2, src, dst)
    t3 = _tc_mid(p2, t2, dinv, W2, b2.reshape(1, -1))
    p3 = _make_agg(32)(t3, src, dst)
    t4 = _tc_mid(p3, t3, dinv, W3, b3.reshape(1, -1))
    p4 = _make_agg(48)(t4, src, dst)
    t5 = _tc_mid(p4, t4, dinv, W4, b4.reshape(1, -1))
    p5 = _make_agg(64)(t5, src, dst)
    t6 = _tc_mid(p5, t5, dinv, W5, b5.reshape(1, -1))
    p6 = _make_agg(96)(t6, src, dst)

    return _tc_final(p6, t6, dinv, batp, W6, b6.reshape(1, 128),
                     Wl1, bl1.reshape(1, 64), Wl2, bl2.reshape(1, 10))
